# baseline (speedup 1.0000x reference)
import functools

import jax
import jax.numpy as jnp
from jax.experimental import pallas as pl
from jax.experimental.pallas import tpu as pltpu

_SLOPE = 0.01
_VMEM_LIMIT = 50 * 1024 * 1024
N_CLASSES = 19


def _conv_body(*args, nin, cins, th, w, pack, fuse, pool):
    xs = args[0:nin]
    ws = args[nin:2 * nin]
    sref = args[2 * nin]
    pref = None
    if fuse:
        owr, osr, oref = args[2 * nin + 1], args[2 * nin + 2], args[2 * nin + 3]
    elif pool:
        oref, pref = args[2 * nin + 1], args[2 * nin + 2]
    else:
        oref = args[2 * nin + 1]
    r0 = pl.program_id(1) * th
    cout = ws[0].shape[1]
    rows = th * w

    acc = jnp.zeros((rows, cout), jnp.float32)
    for xr, wr, c in zip(xs, ws, cins):
        if pack:
            for dy in range(3):
                slab = jnp.concatenate(
                    [xr[0, pl.ds(r0 + dy, th), pl.ds(dx, w), :] for dx in range(3)],
                    axis=-1).reshape(rows, 3 * c)
                acc += jnp.dot(slab, wr[dy * 3 * c:(dy + 1) * 3 * c, :],
                               preferred_element_type=jnp.float32)
        else:
            for dy in range(3):
                for dx in range(3):
                    xt = xr[0, pl.ds(r0 + dy, th), pl.ds(dx, w), :].reshape(rows, c)
                    t = (dy * 3 + dx) * c
                    acc += jnp.dot(xt, wr[t:t + c, :],
                                   preferred_element_type=jnp.float32)
    y = acc + sref[...]
    y = jnp.where(y >= 0.0, y, _SLOPE * y)
    if fuse:
        z = jnp.dot(y.astype(jnp.bfloat16), owr[...],
                    preferred_element_type=jnp.float32) + osr[...]
        oref[0] = z.reshape(th, w, osr.shape[-1]).astype(oref.dtype)
    else:
        yb = y.reshape(th, w, cout).astype(oref.dtype)
        oref[0] = yb
        if pool:
            ph = yb.reshape(th // 2, 2, w, cout).max(axis=1)
            p = ph.reshape(th // 2, w // 2, 2, cout).max(axis=2)
            pref[0] = p


def _halo_chunks(x, nch):
    n, h, w, c = x.shape
    xp = jnp.pad(x, ((0, 0), (1, 1), (1, 1), (0, 0)))
    if nch == 1:
        return xp
    hc = h // nch
    bands = jnp.stack([xp[:, i * hc:i * hc + hc + 2] for i in range(nch)], axis=1)
    return bands.reshape(n * nch, hc + 2, w + 2, c)


def _conv3x3(xs_raw, ws, shift, *, fuse_1x1=None, pool=False,
             out_dtype=jnp.bfloat16):
    n0, h0, w, _ = xs_raw[0].shape
    cins = [xi.shape[-1] for xi in xs_raw]
    cmax = max(cins)
    cout = ws[0].shape[1]
    pack = cmax <= 128
    win_bytes = sum((h0 + 2) * (w + 2) * c * 2 for c in cins)
    nch = 1
    while win_bytes // nch > 6 * 1024 * 1024 and h0 // nch >= 16:
        nch *= 2
    xs = [_halo_chunks(xi, nch) for xi in xs_raw]
    n, hp, wp, _ = xs[0].shape
    h = hp - 2
    rows_t = 2048 if cmax <= 128 else (1024 if cmax <= 256 else 512)
    th = min(h, max(1, rows_t // w))
    num_h = h // th
    nin = len(xs)
    fuse = fuse_1x1 is not None

    in_specs = [pl.BlockSpec((1, hp, wp, xi.shape[-1]), lambda ni, hi: (ni, 0, 0, 0))
                for xi in xs]
    in_specs += [pl.BlockSpec(wi.shape, lambda ni, hi: (0, 0)) for wi in ws]
    in_specs.append(pl.BlockSpec(shift.shape, lambda ni, hi: (0, 0)))
    args = list(xs) + list(ws) + [shift]
    if fuse:
        ow, osv = fuse_1x1
        in_specs += [pl.BlockSpec(ow.shape, lambda ni, hi: (0, 0)),
                     pl.BlockSpec(osv.shape, lambda ni, hi: (0, 0))]
        args += [ow, osv]
        c_final = ow.shape[1]
    else:
        c_final = cout

    body = functools.partial(_conv_body, nin=nin, cins=cins, th=th, w=w,
                             pack=pack, fuse=fuse, pool=pool)
    out_shape = [jax.ShapeDtypeStruct((n, h, w, c_final), out_dtype)]
    out_specs = [pl.BlockSpec((1, th, w, c_final), lambda ni, hi: (ni, hi, 0, 0))]
    if pool:
        out_shape.append(jax.ShapeDtypeStruct((n, h // 2, w // 2, c_final),
                                              out_dtype))
        out_specs.append(pl.BlockSpec((1, th // 2, w // 2, c_final),
                                      lambda ni, hi: (ni, hi, 0, 0)))
    res = pl.pallas_call(
        body,
        out_shape=out_shape,
        grid_spec=pltpu.PrefetchScalarGridSpec(
            num_scalar_prefetch=0,
            grid=(n, num_h),
            in_specs=in_specs,
            out_specs=out_specs,
        ),
        compiler_params=pltpu.CompilerParams(
            dimension_semantics=("parallel", "parallel"),
            vmem_limit_bytes=_VMEM_LIMIT,
        ),
    )(*args)
    out = res[0].reshape(n0, h0, w, res[0].shape[-1])
    if pool:
        return out, res[1].reshape(n0, h0 // 2, w // 2, res[1].shape[-1])
    return out


def _mm_body(x_ref, w_ref, s_ref, o_ref):
    y = jnp.dot(x_ref[...], w_ref[...],
                preferred_element_type=jnp.float32) + s_ref[...]
    y = jnp.where(y >= 0.0, y, _SLOPE * y)
    o_ref[...] = y.astype(o_ref.dtype)


def _entry_conv(x, w2d, shift):
    n, h, w, c = x.shape
    m = n * h * w
    cout = w2d.shape[1]
    xp = jnp.pad(x, ((0, 0), (1, 1), (1, 1), (0, 0)))
    taps = [xp[:, dy:dy + h, dx:dx + w, :] for dy in range(3) for dx in range(3)]
    pat = jnp.stack(taps, axis=3).reshape(m, 9 * c)
    tm = min(m, 4096)
    y = pl.pallas_call(
        _mm_body,
        out_shape=jax.ShapeDtypeStruct((m, cout), jnp.bfloat16),
        grid_spec=pltpu.PrefetchScalarGridSpec(
            num_scalar_prefetch=0,
            grid=(m // tm,),
            in_specs=[pl.BlockSpec((tm, 9 * c), lambda i: (i, 0)),
                      pl.BlockSpec(w2d.shape, lambda i: (0, 0)),
                      pl.BlockSpec(shift.shape, lambda i: (0, 0))],
            out_specs=pl.BlockSpec((tm, cout), lambda i: (i, 0)),
        ),
        compiler_params=pltpu.CompilerParams(
            dimension_semantics=("parallel",),
            vmem_limit_bytes=_VMEM_LIMIT,
        ),
    )(pat, w2d, shift)
    return y.reshape(n, h, w, cout)


def _up2_axis(x, axis):
    s = x.shape[axis]
    m = 2 * s
    pos = jnp.arange(m, dtype=jnp.float32) * ((s - 1) / (m - 1))
    lo = jnp.minimum(jnp.floor(pos).astype(jnp.int32), s - 2)
    t = pos - lo.astype(jnp.float32)
    delta0 = (lo == (jnp.arange(m) // 2))
    shape = [1, 1, 1, 1]
    shape[axis] = m
    t = t.reshape(shape)
    delta0 = delta0.reshape(shape)

    def shift(src, off):
        sl = [slice(None)] * 4
        pad = [slice(None)] * 4
        if off == -1:
            sl[axis] = slice(0, s - 1)
            pad[axis] = slice(0, 1)
            return jnp.concatenate([src[tuple(pad)], src[tuple(sl)]], axis=axis)
        sl[axis] = slice(1, s)
        pad[axis] = slice(s - 1, s)
        return jnp.concatenate([src[tuple(sl)], src[tuple(pad)]], axis=axis)

    y0 = jnp.repeat(x, 2, axis=axis)
    yprev = jnp.repeat(shift(x, -1), 2, axis=axis)
    ynext = jnp.repeat(shift(x, 1), 2, axis=axis)
    a = jnp.where(delta0, y0, yprev).astype(jnp.float32)
    b = jnp.where(delta0, ynext, y0).astype(jnp.float32)
    return a * (1.0 - t) + b * t


def _up2(x):
    y = _up2_axis(x, 1)
    y = _up2_axis(y, 2)
    return y.astype(jnp.bfloat16)


def _split_w(w2d, ca, cb):
    cout = w2d.shape[1]
    w9 = w2d.reshape(9, ca + cb, cout)
    return (w9[:, :ca, :].reshape(9 * ca, cout),
            w9[:, ca:, :].reshape(9 * cb, cout))


def kernel(x, inc_w1, inc_s1, inc_w2, inc_s2,
           down1_w1, down1_s1, down1_w2, down1_s2,
           down2_w1, down2_s1, down2_w2, down2_s2,
           down3_w1, down3_s1, down3_w2, down3_s2,
           down4_w1, down4_s1, down4_w2, down4_s2,
           up1_w1, up1_s1, up1_w2, up1_s2,
           up2_w1, up2_s1, up2_w2, up2_s2,
           up3_w1, up3_s1, up3_w2, up3_s2,
           up4_w1, up4_s1, up4_w2, up4_s2,
           outc_w, outc_s):
    xh = jnp.transpose(x, (0, 2, 3, 1)).astype(jnp.bfloat16)

    t = _entry_conv(xh, inc_w1, inc_s1)
    x1, p = _conv3x3([t], [inc_w2], inc_s2, pool=True)
    t = _conv3x3([p], [down1_w1], down1_s1)
    x2, p = _conv3x3([t], [down1_w2], down1_s2, pool=True)
    t = _conv3x3([p], [down2_w1], down2_s1)
    x3, p = _conv3x3([t], [down2_w2], down2_s2, pool=True)
    t = _conv3x3([p], [down3_w1], down3_s1)
    x4, p = _conv3x3([t], [down3_w2], down3_s2, pool=True)
    t = _conv3x3([p], [down4_w1], down4_s1)
    x5 = _conv3x3([t], [down4_w2], down4_s2)

    def up_in(xlow, skip, w1, s1):
        u = _up2(xlow)
        wa, wb = _split_w(w1, skip.shape[-1], u.shape[-1])
        return _conv3x3([skip, u], [wa, wb], s1)

    y = up_in(x5, x4, up1_w1, up1_s1)
    y = _conv3x3([y], [up1_w2], up1_s2)
    y = up_in(y, x3, up2_w1, up2_s1)
    y = _conv3x3([y], [up2_w2], up2_s2)
    y = up_in(y, x2, up3_w1, up3_s1)
    y = _conv3x3([y], [up3_w2], up3_s2)
    y = up_in(y, x1, up4_w1, up4_s1)

    logits = _conv3x3(
        [y], [up4_w2], up4_s2,
        fuse_1x1=(outc_w[:, :N_CLASSES], outc_s[:, :N_CLASSES]),
        out_dtype=jnp.float32)
    return jnp.transpose(logits, (0, 3, 1, 2))


# speedup vs baseline: 1.3485x; 1.3485x over previous
import functools

import jax
import jax.numpy as jnp
from jax.experimental import pallas as pl
from jax.experimental.pallas import tpu as pltpu

_SLOPE = 0.01
_VMEM_LIMIT = 50 * 1024 * 1024
N_CLASSES = 19


def _conv_body(*args, nin, cins, th, w, pack, fuse, pool):
    xs = args[0:nin]
    ws = args[nin:2 * nin]
    sref = args[2 * nin]
    pref = None
    if fuse:
        owr, osr, oref = args[2 * nin + 1], args[2 * nin + 2], args[2 * nin + 3]
    elif pool:
        oref, pref = args[2 * nin + 1], args[2 * nin + 2]
    else:
        oref = args[2 * nin + 1]
    r0 = pl.program_id(1) * th
    cout = ws[0].shape[1]
    rows = th * w

    acc = jnp.zeros((rows, cout), jnp.float32)
    for xr, wr, c in zip(xs, ws, cins):
        if pack:
            for dy in range(3):
                slab = jnp.concatenate(
                    [xr[0, pl.ds(r0 + dy, th), pl.ds(dx, w), :] for dx in range(3)],
                    axis=-1).reshape(rows, 3 * c)
                acc += jnp.dot(slab, wr[dy * 3 * c:(dy + 1) * 3 * c, :],
                               preferred_element_type=jnp.float32)
        else:
            for dy in range(3):
                for dx in range(3):
                    xt = xr[0, pl.ds(r0 + dy, th), pl.ds(dx, w), :].reshape(rows, c)
                    t = (dy * 3 + dx) * c
                    acc += jnp.dot(xt, wr[t:t + c, :],
                                   preferred_element_type=jnp.float32)
    y = acc + sref[...]
    y = jnp.where(y >= 0.0, y, _SLOPE * y)
    if fuse:
        z = jnp.dot(y.astype(jnp.bfloat16), owr[...],
                    preferred_element_type=jnp.float32) + osr[...]
        oref[0] = z.reshape(th, w, osr.shape[-1]).astype(oref.dtype)
    else:
        yb = y.reshape(th, w, cout).astype(oref.dtype)
        oref[0] = yb
        if pool:
            ph = yb.reshape(th // 2, 2, w, cout).max(axis=1)
            p = ph.reshape(th // 2, w // 2, 2, cout).max(axis=2)
            pref[0] = p


def _halo_chunks(x, nch):
    n, h, w, c = x.shape
    xp = jnp.pad(x, ((0, 0), (1, 1), (1, 1), (0, 0)))
    if nch == 1:
        return xp
    hc = h // nch
    bands = jnp.stack([xp[:, i * hc:i * hc + hc + 2] for i in range(nch)], axis=1)
    return bands.reshape(n * nch, hc + 2, w + 2, c)


def _nch_for(h0, w, cins):
    win_bytes = sum((h0 + 2) * (w + 2) * c * 2 for c in cins)
    nch = 1
    while win_bytes // nch > 6 * 1024 * 1024 and h0 // nch >= 16:
        nch *= 2
    return nch


def _conv_call(xs, cins, n0, h0, w, ws, shift, *, fuse_1x1=None, pool=False,
               out_dtype=jnp.bfloat16):
    cmax = max(cins)
    cout = ws[0].shape[1]
    pack = cmax <= 128
    n, hp, wp, _ = xs[0].shape
    h = hp - 2
    rows_t = 2048 if cmax <= 128 else (1024 if cmax <= 256 else 512)
    th = min(h, max(1, rows_t // w))
    num_h = h // th
    nin = len(xs)
    fuse = fuse_1x1 is not None

    in_specs = [pl.BlockSpec((1, hp, wp, xi.shape[-1]), lambda ni, hi: (ni, 0, 0, 0))
                for xi in xs]
    in_specs += [pl.BlockSpec(wi.shape, lambda ni, hi: (0, 0)) for wi in ws]
    in_specs.append(pl.BlockSpec(shift.shape, lambda ni, hi: (0, 0)))
    args = list(xs) + list(ws) + [shift]
    if fuse:
        ow, osv = fuse_1x1
        in_specs += [pl.BlockSpec(ow.shape, lambda ni, hi: (0, 0)),
                     pl.BlockSpec(osv.shape, lambda ni, hi: (0, 0))]
        args += [ow, osv]
        c_final = ow.shape[1]
    else:
        c_final = cout

    body = functools.partial(_conv_body, nin=nin, cins=cins, th=th, w=w,
                             pack=pack, fuse=fuse, pool=pool)
    out_shape = [jax.ShapeDtypeStruct((n, h, w, c_final), out_dtype)]
    out_specs = [pl.BlockSpec((1, th, w, c_final), lambda ni, hi: (ni, hi, 0, 0))]
    if pool:
        out_shape.append(jax.ShapeDtypeStruct((n, h // 2, w // 2, c_final),
                                              out_dtype))
        out_specs.append(pl.BlockSpec((1, th // 2, w // 2, c_final),
                                      lambda ni, hi: (ni, hi, 0, 0)))
    res = pl.pallas_call(
        body,
        out_shape=out_shape,
        grid_spec=pltpu.PrefetchScalarGridSpec(
            num_scalar_prefetch=0,
            grid=(n, num_h),
            in_specs=in_specs,
            out_specs=out_specs,
        ),
        compiler_params=pltpu.CompilerParams(
            dimension_semantics=("parallel", "parallel"),
            vmem_limit_bytes=_VMEM_LIMIT,
        ),
    )(*args)
    out = res[0].reshape(n0, h0, w, res[0].shape[-1])
    if pool:
        return out, res[1].reshape(n0, h0 // 2, w // 2, res[1].shape[-1])
    return out


def _conv3x3(xs_raw, ws, shift, **kw):
    n0, h0, w, _ = xs_raw[0].shape
    cins = [xi.shape[-1] for xi in xs_raw]
    nch = _nch_for(h0, w, cins)
    xs = [_halo_chunks(xi, nch) for xi in xs_raw]
    return _conv_call(xs, cins, n0, h0, w, ws, shift, **kw)


def _mm_body(x_ref, w_ref, s_ref, o_ref):
    y = jnp.dot(x_ref[...], w_ref[...],
                preferred_element_type=jnp.float32) + s_ref[...]
    y = jnp.where(y >= 0.0, y, _SLOPE * y)
    o_ref[...] = y.astype(o_ref.dtype)


def _entry_conv(x, w2d, shift):
    n, h, w, c = x.shape
    m = n * h * w
    cout = w2d.shape[1]
    xp = jnp.pad(x, ((0, 0), (1, 1), (1, 1), (0, 0)))
    taps = [xp[:, dy:dy + h, dx:dx + w, :] for dy in range(3) for dx in range(3)]
    pat = jnp.stack(taps, axis=3).reshape(m, 9 * c)
    tm = min(m, 4096)
    y = pl.pallas_call(
        _mm_body,
        out_shape=jax.ShapeDtypeStruct((m, cout), jnp.bfloat16),
        grid_spec=pltpu.PrefetchScalarGridSpec(
            num_scalar_prefetch=0,
            grid=(m // tm,),
            in_specs=[pl.BlockSpec((tm, 9 * c), lambda i: (i, 0)),
                      pl.BlockSpec(w2d.shape, lambda i: (0, 0)),
                      pl.BlockSpec(shift.shape, lambda i: (0, 0))],
            out_specs=pl.BlockSpec((tm, cout), lambda i: (i, 0)),
        ),
        compiler_params=pltpu.CompilerParams(
            dimension_semantics=("parallel",),
            vmem_limit_bytes=_VMEM_LIMIT,
        ),
    )(pat, w2d, shift)
    return y.reshape(n, h, w, cout)


def _axis_coeffs(s, shape, axis, parity):
    m = 2 * s
    k = jax.lax.broadcasted_iota(jnp.int32, shape, axis).astype(jnp.float32)
    pos = (2.0 * k + parity) * ((s - 1) / (m - 1))
    kf = jnp.floor(pos)
    lo = jnp.minimum(kf, float(s - 2))
    t = pos - lo
    d0 = lo == k
    return d0, t


def _upb_body(x_ref, o_ref, *, h, w, nch, hc):
    x3 = x_ref[0].astype(jnp.float32)
    c = x3.shape[-1]
    xm = jnp.concatenate([x3[:1], x3[:-1]], axis=0)
    xp = jnp.concatenate([x3[1:], x3[-1:]], axis=0)
    rows = []
    for parity in (0, 1):
        d0, t = _axis_coeffs(h, (h, 1, 1), 0, parity)
        rows.append(jnp.where(d0, (1.0 - t) * x3 + t * xp,
                              (1.0 - t) * xm + t * x3))
    yh = jnp.stack(rows, axis=1).reshape(2 * h, w, c)
    ym = jnp.concatenate([yh[:, :1], yh[:, :-1]], axis=1)
    yp = jnp.concatenate([yh[:, 1:], yh[:, -1:]], axis=1)
    cols = []
    for parity in (0, 1):
        d0, t = _axis_coeffs(w, (1, w, 1), 1, parity)
        cols.append(jnp.where(d0, (1.0 - t) * yh + t * yp,
                              (1.0 - t) * ym + t * yh))
    y = jnp.stack(cols, axis=2).reshape(2 * h, 2 * w, c)
    yb = y.astype(jnp.bfloat16)
    h2, w2 = 2 * h, 2 * w
    zrow = jnp.zeros((1, w2, c), jnp.bfloat16)
    zcol = jnp.zeros((hc + 2, 1, c), jnp.bfloat16)
    bands = []
    for b in range(nch):
        r0 = b * hc
        top = zrow if r0 == 0 else yb[r0 - 1:r0]
        bot = zrow if r0 + hc == h2 else yb[r0 + hc:r0 + hc + 1]
        band = jnp.concatenate([top, yb[r0:r0 + hc], bot], axis=0)
        bands.append(jnp.concatenate([zcol, band, zcol], axis=1))
    o_ref[0] = jnp.stack(bands, axis=0)


def _up2_banded(x, nch):
    n, h, w, c = x.shape
    h2, w2 = 2 * h, 2 * w
    hc = h2 // nch
    body = functools.partial(_upb_body, h=h, w=w, nch=nch, hc=hc)
    out = pl.pallas_call(
        body,
        out_shape=jax.ShapeDtypeStruct((n, nch, hc + 2, w2 + 2, c),
                                       jnp.bfloat16),
        grid_spec=pltpu.PrefetchScalarGridSpec(
            num_scalar_prefetch=0,
            grid=(n,),
            in_specs=[pl.BlockSpec((1, h, w, c), lambda ni: (ni, 0, 0, 0))],
            out_specs=pl.BlockSpec((1, nch, hc + 2, w2 + 2, c),
                                   lambda ni: (ni, 0, 0, 0, 0)),
        ),
        compiler_params=pltpu.CompilerParams(
            dimension_semantics=("parallel",),
            vmem_limit_bytes=_VMEM_LIMIT,
        ),
    )(x)
    return out.reshape(n * nch, hc + 2, w2 + 2, c)


def _split_w(w2d, ca, cb):
    cout = w2d.shape[1]
    w9 = w2d.reshape(9, ca + cb, cout)
    return (w9[:, :ca, :].reshape(9 * ca, cout),
            w9[:, ca:, :].reshape(9 * cb, cout))


def kernel(x, inc_w1, inc_s1, inc_w2, inc_s2,
           down1_w1, down1_s1, down1_w2, down1_s2,
           down2_w1, down2_s1, down2_w2, down2_s2,
           down3_w1, down3_s1, down3_w2, down3_s2,
           down4_w1, down4_s1, down4_w2, down4_s2,
           up1_w1, up1_s1, up1_w2, up1_s2,
           up2_w1, up2_s1, up2_w2, up2_s2,
           up3_w1, up3_s1, up3_w2, up3_s2,
           up4_w1, up4_s1, up4_w2, up4_s2,
           outc_w, outc_s):
    xh = jnp.transpose(x, (0, 2, 3, 1)).astype(jnp.bfloat16)

    t = _entry_conv(xh, inc_w1, inc_s1)
    x1, p = _conv3x3([t], [inc_w2], inc_s2, pool=True)
    t = _conv3x3([p], [down1_w1], down1_s1)
    x2, p = _conv3x3([t], [down1_w2], down1_s2, pool=True)
    t = _conv3x3([p], [down2_w1], down2_s1)
    x3, p = _conv3x3([t], [down2_w2], down2_s2, pool=True)
    t = _conv3x3([p], [down3_w1], down3_s1)
    x4, p = _conv3x3([t], [down3_w2], down3_s2, pool=True)
    t = _conv3x3([p], [down4_w1], down4_s1)
    x5 = _conv3x3([t], [down4_w2], down4_s2)

    def up_in(xlow, skip, w1, s1):
        n0, h0, w, _ = skip.shape
        cins = [skip.shape[-1], xlow.shape[-1]]
        nch = _nch_for(h0, w, cins)
        ub = _up2_banded(xlow, nch)
        wa, wb = _split_w(w1, cins[0], cins[1])
        return _conv_call([_halo_chunks(skip, nch), ub], cins,
                          n0, h0, w, [wa, wb], s1)

    y = up_in(x5, x4, up1_w1, up1_s1)
    y = _conv3x3([y], [up1_w2], up1_s2)
    y = up_in(y, x3, up2_w1, up2_s1)
    y = _conv3x3([y], [up2_w2], up2_s2)
    y = up_in(y, x2, up3_w1, up3_s1)
    y = _conv3x3([y], [up3_w2], up3_s2)
    y = up_in(y, x1, up4_w1, up4_s1)

    logits = _conv3x3(
        [y], [up4_w2], up4_s2,
        fuse_1x1=(outc_w[:, :N_CLASSES], outc_s[:, :N_CLASSES]),
        out_dtype=jnp.float32)
    return jnp.transpose(logits, (0, 3, 1, 2))


# speedup vs baseline: 1.4195x; 1.0526x over previous
import functools

import jax
import jax.numpy as jnp
from jax.experimental import pallas as pl
from jax.experimental.pallas import tpu as pltpu

_SLOPE = 0.01
_VMEM_LIMIT = 50 * 1024 * 1024
N_CLASSES = 19


def _conv_body(*args, nin, cins, th, w, pack, fuse, pool):
    xs = args[0:nin]
    ws = args[nin:2 * nin]
    sref = args[2 * nin]
    pref = None
    if fuse:
        owr, osr, oref = args[2 * nin + 1], args[2 * nin + 2], args[2 * nin + 3]
    elif pool:
        oref, pref = args[2 * nin + 1], args[2 * nin + 2]
    else:
        oref = args[2 * nin + 1]
    r0 = pl.program_id(1) * th
    cout = ws[0].shape[1]
    rows = th * w

    acc = jnp.zeros((rows, cout), jnp.float32)
    for xr, wr, c in zip(xs, ws, cins):
        if pack:
            for dy in range(3):
                slab = jnp.concatenate(
                    [xr[0, pl.ds(r0 + dy, th), pl.ds(dx, w), :] for dx in range(3)],
                    axis=-1).reshape(rows, 3 * c)
                acc += jnp.dot(slab, wr[dy * 3 * c:(dy + 1) * 3 * c, :],
                               preferred_element_type=jnp.float32)
        else:
            for dy in range(3):
                for dx in range(3):
                    xt = xr[0, pl.ds(r0 + dy, th), pl.ds(dx, w), :].reshape(rows, c)
                    t = (dy * 3 + dx) * c
                    acc += jnp.dot(xt, wr[t:t + c, :],
                                   preferred_element_type=jnp.float32)
    y = acc + sref[...]
    y = jnp.where(y >= 0.0, y, _SLOPE * y)
    if fuse:
        z = jnp.dot(y.astype(jnp.bfloat16), owr[...],
                    preferred_element_type=jnp.float32) + osr[...]
        oref[0] = z.reshape(th, w, osr.shape[-1]).astype(oref.dtype)
    else:
        yb = y.reshape(th, w, cout).astype(oref.dtype)
        oref[0] = yb
        if pool:
            ph = yb.reshape(th // 2, 2, w, cout).max(axis=1)
            p = ph.reshape(th // 2, w // 2, 2, cout).max(axis=2)
            pref[0] = p


def _halo_chunks(x, nch):
    n, h, w, c = x.shape
    xp = jnp.pad(x, ((0, 0), (1, 1), (1, 1), (0, 0)))
    if nch == 1:
        return xp
    hc = h // nch
    bands = jnp.stack([xp[:, i * hc:i * hc + hc + 2] for i in range(nch)], axis=1)
    return bands.reshape(n * nch, hc + 2, w + 2, c)


def _nch_for(h0, w, cins):
    win_bytes = sum((h0 + 2) * (w + 2) * c * 2 for c in cins)
    nch = 1
    while win_bytes // nch > 6 * 1024 * 1024 and h0 // nch >= 16:
        nch *= 2
    return nch


def _conv_call(xs, cins, n0, h0, w, ws, shift, *, fuse_1x1=None, pool=False,
               out_dtype=jnp.bfloat16):
    cmax = max(cins)
    cout = ws[0].shape[1]
    pack = cmax <= 128
    n, hp, wp, _ = xs[0].shape
    h = hp - 2
    rows_t = 4096 if cmax <= 128 else (2048 if cmax <= 256 else 1024)
    th = min(h, max(1, rows_t // w))
    num_h = h // th
    nin = len(xs)
    fuse = fuse_1x1 is not None

    in_specs = [pl.BlockSpec((1, hp, wp, xi.shape[-1]), lambda ni, hi: (ni, 0, 0, 0))
                for xi in xs]
    in_specs += [pl.BlockSpec(wi.shape, lambda ni, hi: (0, 0)) for wi in ws]
    in_specs.append(pl.BlockSpec(shift.shape, lambda ni, hi: (0, 0)))
    args = list(xs) + list(ws) + [shift]
    if fuse:
        ow, osv = fuse_1x1
        in_specs += [pl.BlockSpec(ow.shape, lambda ni, hi: (0, 0)),
                     pl.BlockSpec(osv.shape, lambda ni, hi: (0, 0))]
        args += [ow, osv]
        c_final = ow.shape[1]
    else:
        c_final = cout

    body = functools.partial(_conv_body, nin=nin, cins=cins, th=th, w=w,
                             pack=pack, fuse=fuse, pool=pool)
    out_shape = [jax.ShapeDtypeStruct((n, h, w, c_final), out_dtype)]
    out_specs = [pl.BlockSpec((1, th, w, c_final), lambda ni, hi: (ni, hi, 0, 0))]
    if pool:
        out_shape.append(jax.ShapeDtypeStruct((n, h // 2, w // 2, c_final),
                                              out_dtype))
        out_specs.append(pl.BlockSpec((1, th // 2, w // 2, c_final),
                                      lambda ni, hi: (ni, hi, 0, 0)))
    res = pl.pallas_call(
        body,
        out_shape=out_shape,
        grid_spec=pltpu.PrefetchScalarGridSpec(
            num_scalar_prefetch=0,
            grid=(n, num_h),
            in_specs=in_specs,
            out_specs=out_specs,
        ),
        compiler_params=pltpu.CompilerParams(
            dimension_semantics=("parallel", "parallel"),
            vmem_limit_bytes=_VMEM_LIMIT,
        ),
    )(*args)
    out = res[0].reshape(n0, h0, w, res[0].shape[-1])
    if pool:
        return out, res[1].reshape(n0, h0 // 2, w // 2, res[1].shape[-1])
    return out


def _conv3x3(xs_raw, ws, shift, **kw):
    n0, h0, w, _ = xs_raw[0].shape
    cins = [xi.shape[-1] for xi in xs_raw]
    nch = _nch_for(h0, w, cins)
    xs = [_halo_chunks(xi, nch) for xi in xs_raw]
    return _conv_call(xs, cins, n0, h0, w, ws, shift, **kw)


def _mm_body(x_ref, w_ref, s_ref, o_ref):
    y = jnp.dot(x_ref[...], w_ref[...],
                preferred_element_type=jnp.float32) + s_ref[...]
    y = jnp.where(y >= 0.0, y, _SLOPE * y)
    o_ref[...] = y.astype(o_ref.dtype)


def _entry_conv(x, w2d, shift):
    n, h, w, c = x.shape
    m = n * h * w
    cout = w2d.shape[1]
    xp = jnp.pad(x, ((0, 0), (1, 1), (1, 1), (0, 0)))
    taps = [xp[:, dy:dy + h, dx:dx + w, :] for dy in range(3) for dx in range(3)]
    pat = jnp.stack(taps, axis=3).reshape(m, 9 * c)
    tm = min(m, 4096)
    y = pl.pallas_call(
        _mm_body,
        out_shape=jax.ShapeDtypeStruct((m, cout), jnp.bfloat16),
        grid_spec=pltpu.PrefetchScalarGridSpec(
            num_scalar_prefetch=0,
            grid=(m // tm,),
            in_specs=[pl.BlockSpec((tm, 9 * c), lambda i: (i, 0)),
                      pl.BlockSpec(w2d.shape, lambda i: (0, 0)),
                      pl.BlockSpec(shift.shape, lambda i: (0, 0))],
            out_specs=pl.BlockSpec((tm, cout), lambda i: (i, 0)),
        ),
        compiler_params=pltpu.CompilerParams(
            dimension_semantics=("parallel",),
            vmem_limit_bytes=_VMEM_LIMIT,
        ),
    )(pat, w2d, shift)
    return y.reshape(n, h, w, cout)


def _axis_coeffs(s, shape, axis, parity):
    m = 2 * s
    k = jax.lax.broadcasted_iota(jnp.int32, shape, axis).astype(jnp.float32)
    pos = (2.0 * k + parity) * ((s - 1) / (m - 1))
    kf = jnp.floor(pos)
    lo = jnp.minimum(kf, float(s - 2))
    t = pos - lo
    d0 = lo == k
    return d0, t


def _upb_body(x_ref, o_ref, *, h, w, nch, hc):
    x3 = x_ref[0].astype(jnp.float32)
    c = x3.shape[-1]
    xm = jnp.concatenate([x3[:1], x3[:-1]], axis=0)
    xp = jnp.concatenate([x3[1:], x3[-1:]], axis=0)
    rows = []
    for parity in (0, 1):
        d0, t = _axis_coeffs(h, (h, 1, 1), 0, parity)
        rows.append(jnp.where(d0, (1.0 - t) * x3 + t * xp,
                              (1.0 - t) * xm + t * x3))
    yh = jnp.stack(rows, axis=1).reshape(2 * h, w, c)
    ym = jnp.concatenate([yh[:, :1], yh[:, :-1]], axis=1)
    yp = jnp.concatenate([yh[:, 1:], yh[:, -1:]], axis=1)
    cols = []
    for parity in (0, 1):
        d0, t = _axis_coeffs(w, (1, w, 1), 1, parity)
        cols.append(jnp.where(d0, (1.0 - t) * yh + t * yp,
                              (1.0 - t) * ym + t * yh))
    y = jnp.stack(cols, axis=2).reshape(2 * h, 2 * w, c)
    yb = y.astype(jnp.bfloat16)
    h2, w2 = 2 * h, 2 * w
    zrow = jnp.zeros((1, w2, c), jnp.bfloat16)
    zcol = jnp.zeros((hc + 2, 1, c), jnp.bfloat16)
    bands = []
    for b in range(nch):
        r0 = b * hc
        top = zrow if r0 == 0 else yb[r0 - 1:r0]
        bot = zrow if r0 + hc == h2 else yb[r0 + hc:r0 + hc + 1]
        band = jnp.concatenate([top, yb[r0:r0 + hc], bot], axis=0)
        bands.append(jnp.concatenate([zcol, band, zcol], axis=1))
    o_ref[0] = jnp.stack(bands, axis=0)


def _up2_banded(x, nch):
    n, h, w, c = x.shape
    h2, w2 = 2 * h, 2 * w
    hc = h2 // nch
    body = functools.partial(_upb_body, h=h, w=w, nch=nch, hc=hc)
    out = pl.pallas_call(
        body,
        out_shape=jax.ShapeDtypeStruct((n, nch, hc + 2, w2 + 2, c),
                                       jnp.bfloat16),
        grid_spec=pltpu.PrefetchScalarGridSpec(
            num_scalar_prefetch=0,
            grid=(n,),
            in_specs=[pl.BlockSpec((1, h, w, c), lambda ni: (ni, 0, 0, 0))],
            out_specs=pl.BlockSpec((1, nch, hc + 2, w2 + 2, c),
                                   lambda ni: (ni, 0, 0, 0, 0)),
        ),
        compiler_params=pltpu.CompilerParams(
            dimension_semantics=("parallel",),
            vmem_limit_bytes=_VMEM_LIMIT,
        ),
    )(x)
    return out.reshape(n * nch, hc + 2, w2 + 2, c)


def _split_w(w2d, ca, cb):
    cout = w2d.shape[1]
    w9 = w2d.reshape(9, ca + cb, cout)
    return (w9[:, :ca, :].reshape(9 * ca, cout),
            w9[:, ca:, :].reshape(9 * cb, cout))


def kernel(x, inc_w1, inc_s1, inc_w2, inc_s2,
           down1_w1, down1_s1, down1_w2, down1_s2,
           down2_w1, down2_s1, down2_w2, down2_s2,
           down3_w1, down3_s1, down3_w2, down3_s2,
           down4_w1, down4_s1, down4_w2, down4_s2,
           up1_w1, up1_s1, up1_w2, up1_s2,
           up2_w1, up2_s1, up2_w2, up2_s2,
           up3_w1, up3_s1, up3_w2, up3_s2,
           up4_w1, up4_s1, up4_w2, up4_s2,
           outc_w, outc_s):
    xh = jnp.transpose(x, (0, 2, 3, 1)).astype(jnp.bfloat16)

    t = _entry_conv(xh, inc_w1, inc_s1)
    x1, p = _conv3x3([t], [inc_w2], inc_s2, pool=True)
    t = _conv3x3([p], [down1_w1], down1_s1)
    x2, p = _conv3x3([t], [down1_w2], down1_s2, pool=True)
    t = _conv3x3([p], [down2_w1], down2_s1)
    x3, p = _conv3x3([t], [down2_w2], down2_s2, pool=True)
    t = _conv3x3([p], [down3_w1], down3_s1)
    x4, p = _conv3x3([t], [down3_w2], down3_s2, pool=True)
    t = _conv3x3([p], [down4_w1], down4_s1)
    x5 = _conv3x3([t], [down4_w2], down4_s2)

    def up_in(xlow, skip, w1, s1):
        n0, h0, w, _ = skip.shape
        cins = [skip.shape[-1], xlow.shape[-1]]
        nch = _nch_for(h0, w, cins)
        ub = _up2_banded(xlow, nch)
        wa, wb = _split_w(w1, cins[0], cins[1])
        return _conv_call([_halo_chunks(skip, nch), ub], cins,
                          n0, h0, w, [wa, wb], s1)

    y = up_in(x5, x4, up1_w1, up1_s1)
    y = _conv3x3([y], [up1_w2], up1_s2)
    y = up_in(y, x3, up2_w1, up2_s1)
    y = _conv3x3([y], [up2_w2], up2_s2)
    y = up_in(y, x2, up3_w1, up3_s1)
    y = _conv3x3([y], [up3_w2], up3_s2)
    y = up_in(y, x1, up4_w1, up4_s1)

    logits = _conv3x3(
        [y], [up4_w2], up4_s2,
        fuse_1x1=(outc_w[:, :N_CLASSES], outc_s[:, :N_CLASSES]),
        out_dtype=jnp.float32)
    return jnp.transpose(logits, (0, 3, 1, 2))


# speedup vs baseline: 1.4366x; 1.0121x over previous
import functools

import jax
import jax.numpy as jnp
from jax.experimental import pallas as pl
from jax.experimental.pallas import tpu as pltpu

_SLOPE = 0.01
_VMEM_LIMIT = 50 * 1024 * 1024
N_CLASSES = 19


def _conv_body(*args, nin, cins, th, w, pack, fuse, pool):
    xs = args[0:nin]
    ws = args[nin:2 * nin]
    sref = args[2 * nin]
    pref = None
    if fuse:
        owr, osr, oref = args[2 * nin + 1], args[2 * nin + 2], args[2 * nin + 3]
    elif pool:
        oref, pref = args[2 * nin + 1], args[2 * nin + 2]
    else:
        oref = args[2 * nin + 1]
    r0 = pl.program_id(1) * th
    cout = ws[0].shape[1]
    rows = th * w

    acc = jnp.zeros((rows, cout), jnp.float32)
    for xr, wr, c in zip(xs, ws, cins):
        if pack:
            for dy in range(3):
                slab = jnp.concatenate(
                    [xr[0, pl.ds(r0 + dy, th), pl.ds(dx, w), :] for dx in range(3)],
                    axis=-1).reshape(rows, 3 * c)
                acc += jnp.dot(slab, wr[dy * 3 * c:(dy + 1) * 3 * c, :],
                               preferred_element_type=jnp.float32)
        else:
            for dy in range(3):
                for dx in range(3):
                    xt = xr[0, pl.ds(r0 + dy, th), pl.ds(dx, w), :].reshape(rows, c)
                    t = (dy * 3 + dx) * c
                    acc += jnp.dot(xt, wr[t:t + c, :],
                                   preferred_element_type=jnp.float32)
    y = acc + sref[...]
    y = jnp.where(y >= 0.0, y, _SLOPE * y)
    if fuse:
        z = jnp.dot(y.astype(jnp.bfloat16), owr[...],
                    preferred_element_type=jnp.float32) + osr[...]
        oref[0] = z.reshape(th, w, osr.shape[-1]).astype(oref.dtype)
    else:
        yb = y.reshape(th, w, cout).astype(oref.dtype)
        oref[0] = yb
        if pool:
            ph = yb.reshape(th // 2, 2, w, cout).max(axis=1)
            p = ph.reshape(th // 2, w // 2, 2, cout).max(axis=2)
            pref[0] = p


def _halo_chunks(x, nch):
    n, h, w, c = x.shape
    xp = jnp.pad(x, ((0, 0), (1, 1), (1, 1), (0, 0)))
    if nch == 1:
        return xp
    hc = h // nch
    bands = jnp.stack([xp[:, i * hc:i * hc + hc + 2] for i in range(nch)], axis=1)
    return bands.reshape(n * nch, hc + 2, w + 2, c)


def _nch_for(h0, w, cins):
    win_bytes = sum((h0 + 2) * (w + 2) * c * 2 for c in cins)
    nch = 1
    while win_bytes // nch > 6 * 1024 * 1024 and h0 // nch >= 16:
        nch *= 2
    return nch


def _conv_call(xs, cins, n0, h0, w, ws, shift, *, fuse_1x1=None, pool=False,
               out_dtype=jnp.bfloat16):
    cmax = max(cins)
    cout = ws[0].shape[1]
    pack = cmax <= 128
    n, hp, wp, _ = xs[0].shape
    h = hp - 2
    rows_t = 8192 if cmax <= 128 else (4096 if cmax <= 256 else 2048)
    th = min(h, max(1, rows_t // w))
    num_h = h // th
    nin = len(xs)
    fuse = fuse_1x1 is not None

    in_specs = [pl.BlockSpec((1, hp, wp, xi.shape[-1]), lambda ni, hi: (ni, 0, 0, 0))
                for xi in xs]
    in_specs += [pl.BlockSpec(wi.shape, lambda ni, hi: (0, 0)) for wi in ws]
    in_specs.append(pl.BlockSpec(shift.shape, lambda ni, hi: (0, 0)))
    args = list(xs) + list(ws) + [shift]
    if fuse:
        ow, osv = fuse_1x1
        in_specs += [pl.BlockSpec(ow.shape, lambda ni, hi: (0, 0)),
                     pl.BlockSpec(osv.shape, lambda ni, hi: (0, 0))]
        args += [ow, osv]
        c_final = ow.shape[1]
    else:
        c_final = cout

    body = functools.partial(_conv_body, nin=nin, cins=cins, th=th, w=w,
                             pack=pack, fuse=fuse, pool=pool)
    out_shape = [jax.ShapeDtypeStruct((n, h, w, c_final), out_dtype)]
    out_specs = [pl.BlockSpec((1, th, w, c_final), lambda ni, hi: (ni, hi, 0, 0))]
    if pool:
        out_shape.append(jax.ShapeDtypeStruct((n, h // 2, w // 2, c_final),
                                              out_dtype))
        out_specs.append(pl.BlockSpec((1, th // 2, w // 2, c_final),
                                      lambda ni, hi: (ni, hi, 0, 0)))
    res = pl.pallas_call(
        body,
        out_shape=out_shape,
        grid_spec=pltpu.PrefetchScalarGridSpec(
            num_scalar_prefetch=0,
            grid=(n, num_h),
            in_specs=in_specs,
            out_specs=out_specs,
        ),
        compiler_params=pltpu.CompilerParams(
            dimension_semantics=("parallel", "parallel"),
            vmem_limit_bytes=_VMEM_LIMIT,
        ),
    )(*args)
    out = res[0].reshape(n0, h0, w, res[0].shape[-1])
    if pool:
        return out, res[1].reshape(n0, h0 // 2, w // 2, res[1].shape[-1])
    return out


def _conv3x3(xs_raw, ws, shift, **kw):
    n0, h0, w, _ = xs_raw[0].shape
    cins = [xi.shape[-1] for xi in xs_raw]
    nch = _nch_for(h0, w, cins)
    xs = [_halo_chunks(xi, nch) for xi in xs_raw]
    return _conv_call(xs, cins, n0, h0, w, ws, shift, **kw)


def _mm_body(x_ref, w_ref, s_ref, o_ref):
    y = jnp.dot(x_ref[...], w_ref[...],
                preferred_element_type=jnp.float32) + s_ref[...]
    y = jnp.where(y >= 0.0, y, _SLOPE * y)
    o_ref[...] = y.astype(o_ref.dtype)


def _entry_conv(x, w2d, shift):
    n, h, w, c = x.shape
    m = n * h * w
    cout = w2d.shape[1]
    xp = jnp.pad(x, ((0, 0), (1, 1), (1, 1), (0, 0)))
    taps = [xp[:, dy:dy + h, dx:dx + w, :] for dy in range(3) for dx in range(3)]
    pat = jnp.stack(taps, axis=3).reshape(m, 9 * c)
    tm = min(m, 4096)
    y = pl.pallas_call(
        _mm_body,
        out_shape=jax.ShapeDtypeStruct((m, cout), jnp.bfloat16),
        grid_spec=pltpu.PrefetchScalarGridSpec(
            num_scalar_prefetch=0,
            grid=(m // tm,),
            in_specs=[pl.BlockSpec((tm, 9 * c), lambda i: (i, 0)),
                      pl.BlockSpec(w2d.shape, lambda i: (0, 0)),
                      pl.BlockSpec(shift.shape, lambda i: (0, 0))],
            out_specs=pl.BlockSpec((tm, cout), lambda i: (i, 0)),
        ),
        compiler_params=pltpu.CompilerParams(
            dimension_semantics=("parallel",),
            vmem_limit_bytes=_VMEM_LIMIT,
        ),
    )(pat, w2d, shift)
    return y.reshape(n, h, w, cout)


def _axis_coeffs(s, shape, axis, parity):
    m = 2 * s
    k = jax.lax.broadcasted_iota(jnp.int32, shape, axis).astype(jnp.float32)
    pos = (2.0 * k + parity) * ((s - 1) / (m - 1))
    kf = jnp.floor(pos)
    lo = jnp.minimum(kf, float(s - 2))
    t = pos - lo
    d0 = lo == k
    return d0, t


def _upb_body(x_ref, o_ref, *, h, w, nch, hc):
    x3 = x_ref[0].astype(jnp.float32)
    c = x3.shape[-1]
    xm = jnp.concatenate([x3[:1], x3[:-1]], axis=0)
    xp = jnp.concatenate([x3[1:], x3[-1:]], axis=0)
    rows = []
    for parity in (0, 1):
        d0, t = _axis_coeffs(h, (h, 1, 1), 0, parity)
        rows.append(jnp.where(d0, (1.0 - t) * x3 + t * xp,
                              (1.0 - t) * xm + t * x3))
    yh = jnp.stack(rows, axis=1).reshape(2 * h, w, c)
    ym = jnp.concatenate([yh[:, :1], yh[:, :-1]], axis=1)
    yp = jnp.concatenate([yh[:, 1:], yh[:, -1:]], axis=1)
    cols = []
    for parity in (0, 1):
        d0, t = _axis_coeffs(w, (1, w, 1), 1, parity)
        cols.append(jnp.where(d0, (1.0 - t) * yh + t * yp,
                              (1.0 - t) * ym + t * yh))
    y = jnp.stack(cols, axis=2).reshape(2 * h, 2 * w, c)
    yb = y.astype(jnp.bfloat16)
    h2, w2 = 2 * h, 2 * w
    zrow = jnp.zeros((1, w2, c), jnp.bfloat16)
    zcol = jnp.zeros((hc + 2, 1, c), jnp.bfloat16)
    bands = []
    for b in range(nch):
        r0 = b * hc
        top = zrow if r0 == 0 else yb[r0 - 1:r0]
        bot = zrow if r0 + hc == h2 else yb[r0 + hc:r0 + hc + 1]
        band = jnp.concatenate([top, yb[r0:r0 + hc], bot], axis=0)
        bands.append(jnp.concatenate([zcol, band, zcol], axis=1))
    o_ref[0] = jnp.stack(bands, axis=0)


def _up2_banded(x, nch):
    n, h, w, c = x.shape
    h2, w2 = 2 * h, 2 * w
    hc = h2 // nch
    body = functools.partial(_upb_body, h=h, w=w, nch=nch, hc=hc)
    out = pl.pallas_call(
        body,
        out_shape=jax.ShapeDtypeStruct((n, nch, hc + 2, w2 + 2, c),
                                       jnp.bfloat16),
        grid_spec=pltpu.PrefetchScalarGridSpec(
            num_scalar_prefetch=0,
            grid=(n,),
            in_specs=[pl.BlockSpec((1, h, w, c), lambda ni: (ni, 0, 0, 0))],
            out_specs=pl.BlockSpec((1, nch, hc + 2, w2 + 2, c),
                                   lambda ni: (ni, 0, 0, 0, 0)),
        ),
        compiler_params=pltpu.CompilerParams(
            dimension_semantics=("parallel",),
            vmem_limit_bytes=_VMEM_LIMIT,
        ),
    )(x)
    return out.reshape(n * nch, hc + 2, w2 + 2, c)


def _split_w(w2d, ca, cb):
    cout = w2d.shape[1]
    w9 = w2d.reshape(9, ca + cb, cout)
    return (w9[:, :ca, :].reshape(9 * ca, cout),
            w9[:, ca:, :].reshape(9 * cb, cout))


def kernel(x, inc_w1, inc_s1, inc_w2, inc_s2,
           down1_w1, down1_s1, down1_w2, down1_s2,
           down2_w1, down2_s1, down2_w2, down2_s2,
           down3_w1, down3_s1, down3_w2, down3_s2,
           down4_w1, down4_s1, down4_w2, down4_s2,
           up1_w1, up1_s1, up1_w2, up1_s2,
           up2_w1, up2_s1, up2_w2, up2_s2,
           up3_w1, up3_s1, up3_w2, up3_s2,
           up4_w1, up4_s1, up4_w2, up4_s2,
           outc_w, outc_s):
    xh = jnp.transpose(x, (0, 2, 3, 1)).astype(jnp.bfloat16)

    t = _entry_conv(xh, inc_w1, inc_s1)
    x1, p = _conv3x3([t], [inc_w2], inc_s2, pool=True)
    t = _conv3x3([p], [down1_w1], down1_s1)
    x2, p = _conv3x3([t], [down1_w2], down1_s2, pool=True)
    t = _conv3x3([p], [down2_w1], down2_s1)
    x3, p = _conv3x3([t], [down2_w2], down2_s2, pool=True)
    t = _conv3x3([p], [down3_w1], down3_s1)
    x4, p = _conv3x3([t], [down3_w2], down3_s2, pool=True)
    t = _conv3x3([p], [down4_w1], down4_s1)
    x5 = _conv3x3([t], [down4_w2], down4_s2)

    def up_in(xlow, skip, w1, s1):
        n0, h0, w, _ = skip.shape
        cins = [skip.shape[-1], xlow.shape[-1]]
        nch = _nch_for(h0, w, cins)
        ub = _up2_banded(xlow, nch)
        wa, wb = _split_w(w1, cins[0], cins[1])
        return _conv_call([_halo_chunks(skip, nch), ub], cins,
                          n0, h0, w, [wa, wb], s1)

    y = up_in(x5, x4, up1_w1, up1_s1)
    y = _conv3x3([y], [up1_w2], up1_s2)
    y = up_in(y, x3, up2_w1, up2_s1)
    y = _conv3x3([y], [up2_w2], up2_s2)
    y = up_in(y, x2, up3_w1, up3_s1)
    y = _conv3x3([y], [up3_w2], up3_s2)
    y = up_in(y, x1, up4_w1, up4_s1)

    logits = _conv3x3(
        [y], [up4_w2], up4_s2,
        fuse_1x1=(outc_w[:, :N_CLASSES], outc_s[:, :N_CLASSES]),
        out_dtype=jnp.float32)
    return jnp.transpose(logits, (0, 3, 1, 2))
